# baseline (speedup 1.0000x reference)
import jax
import jax.numpy as jnp
from jax import lax
from jax.experimental import pallas as pl
from jax.experimental.pallas import tpu as pltpu
from jax.experimental.pallas import tpu_sc as plsc

_K = 8
_CARD = 2051
_D = 128
_B = 16
_T = 2048
_NC = 2
_NS = 16
_NW = _NC * _NS
_ROWS = _B * _T
_RPW = _ROWS // _NW
_CHUNK = 16
_GROWS = _K * _CHUNK
_NCHUNK = _RPW // _CHUNK
_LANES = 16
_NBUF = 4
_V = _K * _CARD
_SLICE = 1026
_LAST = _CARD - _SLICE


def _body(emb_hbm, idx_hbm, out_hbm, shared, idxraw, idx2, gbuf, obuf,
          ssem, gsem0, gsem1, gsem2, gsem3, osem0, osem1, osem2, osem3):
    cid = lax.axis_index("c")
    sid = lax.axis_index("s")
    wid = cid * _NS + sid
    b = wid // 2
    half = wid % 2
    base = wid * _RPW

    kcb = sid // 2

    @pl.when(sid % 2 == 0)
    def _():
        pltpu.async_copy(emb_hbm.at[kcb, pl.ds(0, _SLICE)],
                         shared.at[pl.ds(kcb * _CARD, _SLICE)], ssem)

    @pl.when(sid % 2 == 1)
    def _():
        pltpu.async_copy(emb_hbm.at[kcb, pl.ds(_SLICE, _LAST)],
                         shared.at[pl.ds(kcb * _CARD + _SLICE, _LAST)], ssem)

    for k in range(_K):
        pltpu.sync_copy(idx_hbm.at[b, k, pl.ds(half * _RPW, _RPW)],
                        idxraw.at[k])

    def build_idx(c, carry):
        for k in range(_K):
            idx2[c, pl.ds(k * _LANES, _LANES)] = (
                idxraw[k, pl.ds(c * _CHUNK, _CHUNK)] + k * _CARD)
        return carry
    lax.fori_loop(0, _NCHUNK, build_idx, 0)

    @pl.when(sid % 2 == 0)
    def _():
        pltpu.make_async_copy(emb_hbm.at[0, pl.ds(0, _SLICE)],
                              shared.at[pl.ds(0, _SLICE)], ssem).wait()

    @pl.when(sid % 2 == 1)
    def _():
        pltpu.make_async_copy(emb_hbm.at[0, pl.ds(0, _LAST)],
                              shared.at[pl.ds(0, _LAST)], ssem).wait()

    plsc.subcore_barrier()

    gsems = (gsem0, gsem1, gsem2, gsem3)
    osems = (osem0, osem1, osem2, osem3)

    def fire_gather(c, s):
        pltpu.async_copy(shared.at[idx2.at[c]], gbuf.at[s], gsems[s])

    def drain_gather(s):
        pltpu.make_async_copy(shared.at[pl.ds(0, _GROWS)], gbuf.at[s],
                              gsems[s]).wait()

    def drain_out(s):
        pltpu.make_async_copy(obuf.at[s], out_hbm.at[pl.ds(base, _CHUNK)],
                              osems[s]).wait()

    for s in range(_NBUF):
        fire_gather(s, s)

    himask = jnp.int32(-65536)

    def outer(g, carry):
        for s in range(_NBUF):
            c = g * _NBUF + s
            drain_gather(s)

            @pl.when(c >= _NBUF)
            def _():
                drain_out(s)

            def sum_rows(r, rc):
                for col in range(_D // 32):
                    ds_ = pl.ds(col * 16, _LANES)
                    ws = [gbuf[s, k * _CHUNK + r, ds_] for k in range(_K)]
                    los = [lax.bitcast_convert_type(lax.shift_left(w, 16),
                                                    jnp.float32) for w in ws]
                    his = [lax.bitcast_convert_type(w & himask, jnp.float32)
                           for w in ws]
                    lo = ((los[0] + los[1]) + (los[2] + los[3])) + \
                         ((los[4] + los[5]) + (los[6] + los[7]))
                    hi = ((his[0] + his[1]) + (his[2] + his[3])) + \
                         ((his[4] + his[5]) + (his[6] + his[7]))
                    obuf[s, r, pl.ds(col * 32, _LANES)] = lo
                    obuf[s, r, pl.ds(col * 32 + _LANES, _LANES)] = hi
                return rc
            lax.fori_loop(0, _CHUNK, sum_rows, 0)

            pltpu.async_copy(obuf.at[s],
                             out_hbm.at[pl.ds(base + c * _CHUNK, _CHUNK)],
                             osems[s])

            @pl.when(c + _NBUF < _NCHUNK)
            def _():
                fire_gather(c + _NBUF, s)
        return carry
    lax.fori_loop(0, _NCHUNK // _NBUF, outer, 0)

    for s in range(_NBUF):
        drain_out(s)


@jax.jit
def _embed_sum(emb_pk, idx3d):
    mesh = plsc.VectorSubcoreMesh(core_axis_name="c", subcore_axis_name="s")
    kfn = pl.kernel(
        _body,
        out_type=jax.ShapeDtypeStruct((_ROWS, _D), jnp.float32),
        mesh=mesh,
        compiler_params=pltpu.CompilerParams(use_tc_tiling_on_sc=False),
        scratch_types=[
            pltpu.VMEM_SHARED((_V, _D // 2), jnp.int32),
            pltpu.VMEM((_K, _RPW), jnp.int32),
            pltpu.VMEM((_NCHUNK, _GROWS), jnp.int32),
            pltpu.VMEM((_NBUF, _GROWS, _D // 2), jnp.int32),
            pltpu.VMEM((_NBUF, _CHUNK, _D), jnp.float32),
            pltpu.SemaphoreType.DMA,
            pltpu.SemaphoreType.DMA,
            pltpu.SemaphoreType.DMA,
            pltpu.SemaphoreType.DMA,
            pltpu.SemaphoreType.DMA,
            pltpu.SemaphoreType.DMA,
            pltpu.SemaphoreType.DMA,
            pltpu.SemaphoreType.DMA,
            pltpu.SemaphoreType.DMA,
        ],
    )
    return kfn(emb_pk, idx3d)


def kernel(indices, emb):
    idx3d = indices.astype(jnp.int32)
    emb_u = lax.bitcast_convert_type(emb, jnp.uint32)
    rnd = (emb_u + 0x7FFF + ((emb_u >> 16) & 1)) >> 16
    pk = lax.bitcast_convert_type(
        jnp.concatenate(
            [rnd[:, :, g * 32:g * 32 + 16] | (rnd[:, :, g * 32 + 16:(g + 1) * 32] << 16)
             for g in range(_D // 32)], axis=2),
        jnp.int32)
    out = _embed_sum(pk, idx3d)
    return out.reshape(_B, _T, _D)


# speedup vs baseline: 1.3929x; 1.3929x over previous
import jax
import jax.numpy as jnp
from jax import lax
from jax.experimental import pallas as pl
from jax.experimental.pallas import tpu as pltpu
from jax.experimental.pallas import tpu_sc as plsc

_K = 8
_CARD = 2051
_D = 128
_B = 16
_T = 2048
_NC = 2
_NS = 16
_NW = _NC * _NS
_ROWS = _B * _T
_RPW = _ROWS // _NW
_CHUNK = 16
_GROWS = _K * _CHUNK
_NCHUNK = _RPW // _CHUNK
_LANES = 16
_NBUF = 4
_V = _K * _CARD
_SLICE = 1026
_LAST = _CARD - _SLICE


def _body(emb_hbm, idx_hbm, out_hbm, shared, idxraw, idx2, gbuf, obuf,
          ssem, gsem0, gsem1, gsem2, gsem3, osem0, osem1, osem2, osem3):
    cid = lax.axis_index("c")
    sid = lax.axis_index("s")
    wid = cid * _NS + sid
    b = wid // 2
    half = wid % 2
    base = wid * _RPW

    kcb = sid // 2

    @pl.when(sid % 2 == 0)
    def _():
        pltpu.async_copy(emb_hbm.at[kcb, pl.ds(0, _SLICE)],
                         shared.at[pl.ds(kcb * _CARD, _SLICE)], ssem)

    @pl.when(sid % 2 == 1)
    def _():
        pltpu.async_copy(emb_hbm.at[kcb, pl.ds(_SLICE, _LAST)],
                         shared.at[pl.ds(kcb * _CARD + _SLICE, _LAST)], ssem)

    for k in range(_K):
        pltpu.sync_copy(idx_hbm.at[b, k, pl.ds(half * _RPW, _RPW)],
                        idxraw.at[k])

    def build_idx(c, carry):
        for k in range(_K):
            idx2[c, pl.ds(k * _LANES, _LANES)] = (
                idxraw[k, pl.ds(c * _CHUNK, _CHUNK)] + k * _CARD)
        return carry
    lax.fori_loop(0, _NCHUNK, build_idx, 0)

    @pl.when(sid % 2 == 0)
    def _():
        pltpu.make_async_copy(emb_hbm.at[0, pl.ds(0, _SLICE)],
                              shared.at[pl.ds(0, _SLICE)], ssem).wait()

    @pl.when(sid % 2 == 1)
    def _():
        pltpu.make_async_copy(emb_hbm.at[0, pl.ds(0, _LAST)],
                              shared.at[pl.ds(0, _LAST)], ssem).wait()

    plsc.subcore_barrier()

    gsems = (gsem0, gsem1, gsem2, gsem3)
    osems = (osem0, osem1, osem2, osem3)

    def fire_gather(c, s):
        pltpu.async_copy(shared.at[idx2.at[c]], gbuf.at[s], gsems[s])

    def drain_gather(s):
        pltpu.make_async_copy(shared.at[pl.ds(0, _GROWS)], gbuf.at[s],
                              gsems[s]).wait()

    def drain_out(s):
        pltpu.make_async_copy(obuf.at[s], out_hbm.at[pl.ds(base, _CHUNK)],
                              osems[s]).wait()

    for s in range(_NBUF):
        fire_gather(s, s)

    himask = jnp.int32(-65536)

    def outer(g, carry):
        for s in range(_NBUF):
            c = g * _NBUF + s
            drain_gather(s)

            @pl.when(c >= _NBUF)
            def _():
                drain_out(s)

            def sum_rows(r, rc):
                for col in range(_D // 32):
                    ds_ = pl.ds(col * 16, _LANES)
                    ws = [gbuf[s, k * _CHUNK + r, ds_] for k in range(_K)]
                    los = [lax.bitcast_convert_type(lax.shift_left(w, 16),
                                                    jnp.float32) for w in ws]
                    his = [lax.bitcast_convert_type(w & himask, jnp.float32)
                           for w in ws]
                    lo = ((los[0] + los[1]) + (los[2] + los[3])) + \
                         ((los[4] + los[5]) + (los[6] + los[7]))
                    hi = ((his[0] + his[1]) + (his[2] + his[3])) + \
                         ((his[4] + his[5]) + (his[6] + his[7]))
                    obuf[s, r, pl.ds(col * 16, _LANES)] = lo
                    obuf[s, r, pl.ds(64 + col * 16, _LANES)] = hi
                return rc
            lax.fori_loop(0, _CHUNK, sum_rows, 0)

            pltpu.async_copy(obuf.at[s],
                             out_hbm.at[pl.ds(base + c * _CHUNK, _CHUNK)],
                             osems[s])

            @pl.when(c + _NBUF < _NCHUNK)
            def _():
                fire_gather(c + _NBUF, s)
        return carry
    lax.fori_loop(0, _NCHUNK // _NBUF, outer, 0)

    for s in range(_NBUF):
        drain_out(s)


@jax.jit
def _embed_sum(emb_pk, idx3d):
    mesh = plsc.VectorSubcoreMesh(core_axis_name="c", subcore_axis_name="s")
    kfn = pl.kernel(
        _body,
        out_type=jax.ShapeDtypeStruct((_ROWS, _D), jnp.float32),
        mesh=mesh,
        compiler_params=pltpu.CompilerParams(use_tc_tiling_on_sc=False),
        scratch_types=[
            pltpu.VMEM_SHARED((_V, _D // 2), jnp.int32),
            pltpu.VMEM((_K, _RPW), jnp.int32),
            pltpu.VMEM((_NCHUNK, _GROWS), jnp.int32),
            pltpu.VMEM((_NBUF, _GROWS, _D // 2), jnp.int32),
            pltpu.VMEM((_NBUF, _CHUNK, _D), jnp.float32),
            pltpu.SemaphoreType.DMA,
            pltpu.SemaphoreType.DMA,
            pltpu.SemaphoreType.DMA,
            pltpu.SemaphoreType.DMA,
            pltpu.SemaphoreType.DMA,
            pltpu.SemaphoreType.DMA,
            pltpu.SemaphoreType.DMA,
            pltpu.SemaphoreType.DMA,
            pltpu.SemaphoreType.DMA,
        ],
    )
    return kfn(emb_pk, idx3d)


def kernel(indices, emb):
    idx3d = indices.astype(jnp.int32)
    emb_u = lax.bitcast_convert_type(emb, jnp.uint32)
    rnd = (emb_u + 0x7FFF + ((emb_u >> 16) & 1)) >> 16
    pk = lax.bitcast_convert_type(
        rnd[:, :, :_D // 2] | (rnd[:, :, _D // 2:] << 16), jnp.int32)
    out = _embed_sum(pk, idx3d)
    return out.reshape(_B, _T, _D)
